# Initial kernel scaffold; baseline (speedup 1.0000x reference)
import jax
import jax.numpy as jnp
from jax.experimental import pallas as pl


def kernel(x, edge_index, W0, b0, g0, bt0, W1, b1, g1, bt1, W2, b2, g2, bt2, W3, b3, g3, bt3, W4, b4, g4, bt4, Wlin, blin):
    raise NotImplementedError("write your pallas kernel here")


# speedup vs baseline: 11.4742x; 11.4742x over previous
import functools

import jax
import jax.numpy as jnp
from jax import lax
from jax.experimental import pallas as pl
from jax.experimental.pallas import tpu as pltpu
from jax.experimental.pallas import tpu_sc as plsc

_N = 50000
_E = 1600000
_R = 1568
_NPAD = 50176
_G = _NPAD // _R
_CH = 2048
_SUB = _CH // 128
_EPAD = 1638400
_EPT = _EPAD // 32
_NCH = _EPT // _CH
_ROWS_PT = _NPAD // 16
_EPS = 1e-5


@functools.lru_cache(maxsize=None)
def _make_agg(C):
    mesh = plsc.VectorSubcoreMesh(core_axis_name="c", subcore_axis_name="s")

    @functools.partial(
        pl.kernel,
        mesh=mesh,
        compiler_params=pltpu.CompilerParams(use_tc_tiling_on_sc=False),
        out_type=jax.ShapeDtypeStruct((2, _NPAD, C), jnp.float32),
        scratch_types=[
            pltpu.VMEM((_SUB, 128), jnp.int32),
            pltpu.VMEM((_SUB, 128), jnp.int32),
            pltpu.VMEM((_CH, C), jnp.float32),
            pltpu.VMEM_SHARED((_NPAD, C), jnp.float32),
            pltpu.SemaphoreType.DMA,
            pltpu.SemaphoreType.DMA,
        ],
    )
    def agg(u_hbm, src_hbm, dst_hbm, zero_hbm, out_hbm,
            src_v, dst_v, rows_v, acc, gsem, ssem):
        c = lax.axis_index("c")
        s = lax.axis_index("s")
        zbase = s * _ROWS_PT
        pltpu.sync_copy(zero_hbm.at[pl.ds(zbase, _ROWS_PT)],
                        acc.at[pl.ds(zbase, _ROWS_PT)])
        plsc.subcore_barrier()

        tile_row0 = (c * 16 + s) * (_EPT // 128)

        def chunk(ch, carry):
            rb = tile_row0 + ch * _SUB
            pltpu.sync_copy(src_hbm.at[pl.ds(rb, _SUB)], src_v)
            pltpu.sync_copy(dst_hbm.at[pl.ds(rb, _SUB)], dst_v)
            gets = []
            for j in range(_SUB):
                gets.append(pltpu.async_copy(
                    u_hbm.at[src_v.at[j]],
                    rows_v.at[pl.ds(j * 128, 128)], gsem))
            for g in gets:
                g.wait()
            puts = []
            for j in range(_SUB):
                puts.append(pltpu.async_copy(
                    rows_v.at[pl.ds(j * 128, 128)],
                    acc.at[dst_v.at[j]], ssem, add=True))
            for p in puts:
                p.wait()
            return carry

        lax.fori_loop(0, _NCH, chunk, 0)
        plsc.subcore_barrier()
        pltpu.sync_copy(acc.at[pl.ds(zbase, _ROWS_PT)],
                        out_hbm.at[c, pl.ds(zbase, _ROWS_PT)])

    return agg


def _row(C):
    return pl.BlockSpec((_R, C), lambda i: (i, 0))


def _prow(C):
    return pl.BlockSpec((2, _R, C), lambda i: (0, i, 0))


def _full(shape):
    return pl.BlockSpec(shape, lambda i: (0,) * len(shape))


def _rows_iota(i):
    return lax.broadcasted_iota(jnp.int32, (_R, 1), 0) + i * _R


def _sums_update(i, pre, sums_ref, F):
    m = _rows_iota(i) < _N
    pm = jnp.where(m, pre, 0.0)
    s = jnp.concatenate(
        [jnp.sum(pm, axis=0, keepdims=True),
         jnp.sum(pm * pm, axis=0, keepdims=True),
         jnp.zeros((6, F), jnp.float32)], axis=0)

    @pl.when(i == 0)
    def _():
        sums_ref[...] = s

    @pl.when(i > 0)
    def _():
        sums_ref[...] += s


def _bn_leaky(pre, sums_ref, g_ref, bt_ref):
    s1 = sums_ref[0:1, :]
    s2 = sums_ref[1:2, :]
    mean = s1 / _N
    var = s2 / _N - mean * mean
    alpha = g_ref[...] * lax.rsqrt(var + _EPS)
    beta = bt_ref[...] - mean * alpha
    h = alpha * pre + beta
    return jnp.where(h >= 0, h, 0.01 * h)


def _disk_call(cnt_parts, x_p):
    def body(cnt_ref, x_ref, dis_ref, u0_ref):
        i = pl.program_id(0)
        c = cnt_ref[0] + cnt_ref[1]
        dis = jnp.where(_rows_iota(i) < _N, lax.rsqrt(c + 1.0), 0.0)
        dis_ref[...] = dis
        u0_ref[...] = dis * x_ref[...]

    return pl.pallas_call(
        body,
        grid=(_G,),
        in_specs=[_prow(1), _row(8)],
        out_specs=[_row(1), _row(8)],
        out_shape=[jax.ShapeDtypeStruct((_NPAD, 1), jnp.float32),
                   jax.ShapeDtypeStruct((_NPAD, 8), jnp.float32)],
    )(cnt_parts, x_p)


def _q_call(p_list, u_list, dis, b, W=None):
    n = len(u_list)
    CO = u_list[0].shape[1]
    F = W.shape[1] if W is not None else n * CO

    def body(*refs):
        p_refs = refs[:n]
        u_refs = refs[n:2 * n]
        dis_ref = refs[2 * n]
        b_ref = refs[2 * n + 1]
        W_ref = refs[2 * n + 2] if W is not None else None
        pre_ref, sums_ref = refs[-2:]
        i = pl.program_id(0)
        d = dis_ref[...]
        ts = [d * (p[0] + p[1] + u[...]) for p, u in zip(p_refs, u_refs)]
        t = ts[0] if n == 1 else jnp.concatenate(ts, axis=1)
        if W is not None:
            pre = jnp.dot(t, W_ref[...], preferred_element_type=jnp.float32)
        else:
            pre = t
        pre = pre + b_ref[...]
        pre_ref[...] = pre
        _sums_update(i, pre, sums_ref, F)

    in_specs = [_prow(CO)] * n + [_row(CO)] * n + [_row(1), _full((1, F))]
    args = list(p_list) + list(u_list) + [dis, b]
    if W is not None:
        in_specs.append(_full(W.shape))
        args.append(W)
    return pl.pallas_call(
        body,
        grid=(_G,),
        in_specs=in_specs,
        out_specs=[_row(F), _full((8, F))],
        out_shape=[jax.ShapeDtypeStruct((_NPAD, F), jnp.float32),
                   jax.ShapeDtypeStruct((8, F), jnp.float32)],
    )(*args)


def _p_call(pre, sums, g, bt, dis, CO, W=None):
    F = pre.shape[1]
    FO = W.shape[1] if W is not None else F
    nsplit = FO // CO

    def body(*refs):
        pre_ref, sums_ref, g_ref, bt_ref, dis_ref = refs[:5]
        W_ref = refs[5] if W is not None else None
        u_refs = refs[-nsplit:]
        h = _bn_leaky(pre_ref[...], sums_ref, g_ref, bt_ref)
        if W is not None:
            h = jnp.dot(h, W_ref[...], preferred_element_type=jnp.float32)
        u = dis_ref[...] * h
        for k, ur in enumerate(u_refs):
            ur[...] = u[:, k * CO:(k + 1) * CO]

    in_specs = [_row(F), _full((8, F)), _full((1, F)), _full((1, F)), _row(1)]
    args = [pre, sums, g, bt, dis]
    if W is not None:
        in_specs.append(_full(W.shape))
        args.append(W)
    return pl.pallas_call(
        body,
        grid=(_G,),
        in_specs=in_specs,
        out_specs=[_row(CO)] * nsplit,
        out_shape=[jax.ShapeDtypeStruct((_NPAD, CO), jnp.float32)] * nsplit,
    )(*args)


def _fin_call(pre, sums, g, bt, Wl, bl):
    F = pre.shape[1]
    FO = Wl.shape[1]

    def body(pre_ref, sums_ref, g_ref, bt_ref, W_ref, b_ref, out_ref):
        h = _bn_leaky(pre_ref[...], sums_ref, g_ref, bt_ref)
        out_ref[...] = jnp.dot(
            h, W_ref[...], preferred_element_type=jnp.float32) + b_ref[...]

    return pl.pallas_call(
        body,
        grid=(_G,),
        in_specs=[_row(F), _full((8, F)), _full((1, F)), _full((1, F)),
                  _full((F, FO)), _full((1, FO))],
        out_specs=_row(FO),
        out_shape=jax.ShapeDtypeStruct((_NPAD, FO), jnp.float32),
    )(pre, sums, g, bt, Wl, bl)


def kernel(x, edge_index, W0, b0, g0, bt0, W1, b1, g1, bt1, W2, b2, g2, bt2,
           W3, b3, g3, bt3, W4, b4, g4, bt4, Wlin, blin):
    f32 = jnp.float32
    src = edge_index[0]
    dst = edge_index[1]
    padlen = _EPAD - _E
    pad = jnp.full((padlen,), _N, jnp.int32)
    src2 = jnp.concatenate([src, pad]).reshape(_EPAD // 128, 128)
    dst2 = jnp.concatenate([dst, pad]).reshape(_EPAD // 128, 128)

    x_p = jnp.zeros((_NPAD, 8), f32).at[:_N, :6].set(x)
    node_mask = (jnp.arange(_NPAD) < _N).astype(f32)[:, None]
    zeros = {C: jnp.zeros((_NPAD, C), f32) for C in (1, 8, 16)}

    W0p = jnp.zeros((8, 32), f32).at[:6].set(W0)
    Wlp = jnp.zeros((16, 8), f32).at[:, :3].set(Wlin)
    blp = jnp.zeros((1, 8), f32).at[0, :3].set(blin)
    r2 = lambda v: v.reshape(1, -1)

    def agg(u_chunks, C):
        k = _make_agg(C)
        return [k(u, src2, dst2, zeros[C]) for u in u_chunks]

    cnt, = agg([node_mask], 1)
    dis, u0 = _disk_call(cnt, x_p)

    pre, sums = _q_call(agg([u0], 8), [u0], dis, r2(b0), W0p)
    u1 = _p_call(pre, sums, r2(g0), r2(bt0), dis, 16)

    pre, sums = _q_call(agg(u1, 16), u1, dis, r2(b1), W1)
    u2 = _p_call(pre, sums, r2(g1), r2(bt1), dis, 16)

    pre, sums = _q_call(agg(u2, 16), u2, dis, r2(b2), W2)

    u3 = _p_call(pre, sums, r2(g2), r2(bt2), dis, 16, W3)
    pre, sums = _q_call(agg(u3, 16), u3, dis, r2(b3))

    u4 = _p_call(pre, sums, r2(g3), r2(bt3), dis, 16, W4)
    pre, sums = _q_call(agg(u4, 16), u4, dis, r2(b4))

    out = _fin_call(pre, sums, r2(g4), r2(bt4), Wlp, blp)
    return out[:_N, :3]


# speedup vs baseline: 14.4484x; 1.2592x over previous
import functools

import jax
import jax.numpy as jnp
from jax import lax
from jax.experimental import pallas as pl
from jax.experimental.pallas import tpu as pltpu
from jax.experimental.pallas import tpu_sc as plsc

_N = 50000
_E = 1600000
_R = 1568
_NPAD = 50176
_G = _NPAD // _R
_CH = 2048
_SUB = _CH // 128
_EPAD = 1638400
_EPT = _EPAD // 32
_NCH = _EPT // _CH
_ROWS_PT = _NPAD // 16
_EPS = 1e-5


_CHUNK_FOR_C = {1: 2048, 8: 2048, 16: 1024, 32: 256}


@functools.lru_cache(maxsize=None)
def _make_agg(C):
    CH = _CHUNK_FOR_C[C]
    SUB = CH // 128
    NCH = _EPT // CH
    NG = NCH // 2
    mesh = plsc.VectorSubcoreMesh(core_axis_name="c", subcore_axis_name="s")

    @functools.partial(
        pl.kernel,
        mesh=mesh,
        compiler_params=pltpu.CompilerParams(use_tc_tiling_on_sc=False),
        out_type=jax.ShapeDtypeStruct((2, _NPAD, C), jnp.float32),
        scratch_types=[
            pltpu.VMEM((SUB, 128), jnp.int32),
            pltpu.VMEM((SUB, 128), jnp.int32),
            pltpu.VMEM((SUB, 128), jnp.int32),
            pltpu.VMEM((SUB, 128), jnp.int32),
            pltpu.VMEM((CH, C), jnp.float32),
            pltpu.VMEM((CH, C), jnp.float32),
            pltpu.VMEM_SHARED((_NPAD, C), jnp.float32),
            pltpu.SemaphoreType.DMA,
            pltpu.SemaphoreType.DMA,
            pltpu.SemaphoreType.DMA,
            pltpu.SemaphoreType.DMA,
        ],
    )
    def agg(u_hbm, src_hbm, dst_hbm, zero_hbm, out_hbm,
            srcA, dstA, srcB, dstB, rows0, rows1, acc,
            gsem0, gsem1, ssem0, ssem1):
        c = lax.axis_index("c")
        s = lax.axis_index("s")
        zbase = s * _ROWS_PT
        pltpu.sync_copy(zero_hbm.at[pl.ds(zbase, _ROWS_PT)],
                        acc.at[pl.ds(zbase, _ROWS_PT)])
        plsc.subcore_barrier()

        tile_row0 = (c * 16 + s) * (_EPT // 128)

        def load(ch, src_v, dst_v):
            rb = tile_row0 + ch * SUB
            pltpu.sync_copy(src_hbm.at[pl.ds(rb, SUB)], src_v)
            pltpu.sync_copy(dst_hbm.at[pl.ds(rb, SUB)], dst_v)

        def fire_g(src_v, rows, sem):
            for j in range(SUB):
                pltpu.async_copy(u_hbm.at[src_v.at[j]],
                                 rows.at[pl.ds(j * 128, 128)], sem)

        def drain_g(src_v, rows, sem):
            for j in range(SUB):
                pltpu.make_async_copy(u_hbm.at[src_v.at[j]],
                                      rows.at[pl.ds(j * 128, 128)], sem).wait()

        def fire_s(dst_v, rows, sem):
            for j in range(SUB):
                pltpu.async_copy(rows.at[pl.ds(j * 128, 128)],
                                 acc.at[dst_v.at[j]], sem, add=True)

        def drain_s(dst_v, rows, sem):
            for j in range(SUB):
                pltpu.make_async_copy(rows.at[pl.ds(j * 128, 128)],
                                      acc.at[dst_v.at[j]], sem).wait()

        load(0, srcA, dstA)
        fire_g(srcA, rows0, gsem0)

        def pair(g, carry):
            o = 2 * g + 1
            load(o, srcB, dstB)
            fire_g(srcB, rows1, gsem1)
            drain_g(srcA, rows0, gsem0)
            fire_s(dstA, rows0, ssem0)
            drain_s(dstA, rows0, ssem0)

            @pl.when(g + 1 < NG)
            def _():
                load(o + 1, srcA, dstA)
                fire_g(srcA, rows0, gsem0)

            drain_g(srcB, rows1, gsem1)
            fire_s(dstB, rows1, ssem1)
            drain_s(dstB, rows1, ssem1)
            return carry

        lax.fori_loop(0, NG, pair, 0)
        plsc.subcore_barrier()
        pltpu.sync_copy(acc.at[pl.ds(zbase, _ROWS_PT)],
                        out_hbm.at[c, pl.ds(zbase, _ROWS_PT)])

    return agg


def _row(C):
    return pl.BlockSpec((_R, C), lambda i: (i, 0))


def _prow(C):
    return pl.BlockSpec((2, _R, C), lambda i: (0, i, 0))


def _full(shape):
    return pl.BlockSpec(shape, lambda i: (0,) * len(shape))


def _rows_iota(i):
    return lax.broadcasted_iota(jnp.int32, (_R, 1), 0) + i * _R


def _sums_update(i, pre, sums_ref, F):
    m = _rows_iota(i) < _N
    pm = jnp.where(m, pre, 0.0)
    s = jnp.concatenate(
        [jnp.sum(pm, axis=0, keepdims=True),
         jnp.sum(pm * pm, axis=0, keepdims=True),
         jnp.zeros((6, F), jnp.float32)], axis=0)

    @pl.when(i == 0)
    def _():
        sums_ref[...] = s

    @pl.when(i > 0)
    def _():
        sums_ref[...] += s


def _bn_leaky(pre, sums_ref, g_ref, bt_ref):
    s1 = sums_ref[0:1, :]
    s2 = sums_ref[1:2, :]
    mean = s1 / _N
    var = s2 / _N - mean * mean
    alpha = g_ref[...] * lax.rsqrt(var + _EPS)
    beta = bt_ref[...] - mean * alpha
    h = alpha * pre + beta
    return jnp.where(h >= 0, h, 0.01 * h)


def _disk_call(cnt_parts, x_p):
    def body(cnt_ref, x_ref, dis_ref, u0_ref):
        i = pl.program_id(0)
        c = cnt_ref[0] + cnt_ref[1]
        dis = jnp.where(_rows_iota(i) < _N, lax.rsqrt(c + 1.0), 0.0)
        dis_ref[...] = dis
        u0_ref[...] = dis * x_ref[...]

    return pl.pallas_call(
        body,
        grid=(_G,),
        in_specs=[_prow(1), _row(8)],
        out_specs=[_row(1), _row(8)],
        out_shape=[jax.ShapeDtypeStruct((_NPAD, 1), jnp.float32),
                   jax.ShapeDtypeStruct((_NPAD, 8), jnp.float32)],
    )(cnt_parts, x_p)


def _q_call(p_list, u_list, dis, b, W=None):
    n = len(u_list)
    CO = u_list[0].shape[1]
    F = W.shape[1] if W is not None else n * CO

    def body(*refs):
        p_refs = refs[:n]
        u_refs = refs[n:2 * n]
        dis_ref = refs[2 * n]
        b_ref = refs[2 * n + 1]
        W_ref = refs[2 * n + 2] if W is not None else None
        pre_ref, sums_ref = refs[-2:]
        i = pl.program_id(0)
        d = dis_ref[...]
        ts = [d * (p[0] + p[1] + u[...]) for p, u in zip(p_refs, u_refs)]
        t = ts[0] if n == 1 else jnp.concatenate(ts, axis=1)
        if W is not None:
            pre = jnp.dot(t, W_ref[...], preferred_element_type=jnp.float32)
        else:
            pre = t
        pre = pre + b_ref[...]
        pre_ref[...] = pre
        _sums_update(i, pre, sums_ref, F)

    in_specs = [_prow(CO)] * n + [_row(CO)] * n + [_row(1), _full((1, F))]
    args = list(p_list) + list(u_list) + [dis, b]
    if W is not None:
        in_specs.append(_full(W.shape))
        args.append(W)
    return pl.pallas_call(
        body,
        grid=(_G,),
        in_specs=in_specs,
        out_specs=[_row(F), _full((8, F))],
        out_shape=[jax.ShapeDtypeStruct((_NPAD, F), jnp.float32),
                   jax.ShapeDtypeStruct((8, F), jnp.float32)],
    )(*args)


def _p_call(pre, sums, g, bt, dis, CO, W=None):
    F = pre.shape[1]
    FO = W.shape[1] if W is not None else F
    nsplit = FO // CO

    def body(*refs):
        pre_ref, sums_ref, g_ref, bt_ref, dis_ref = refs[:5]
        W_ref = refs[5] if W is not None else None
        u_refs = refs[-nsplit:]
        h = _bn_leaky(pre_ref[...], sums_ref, g_ref, bt_ref)
        if W is not None:
            h = jnp.dot(h, W_ref[...], preferred_element_type=jnp.float32)
        u = dis_ref[...] * h
        for k, ur in enumerate(u_refs):
            ur[...] = u[:, k * CO:(k + 1) * CO]

    in_specs = [_row(F), _full((8, F)), _full((1, F)), _full((1, F)), _row(1)]
    args = [pre, sums, g, bt, dis]
    if W is not None:
        in_specs.append(_full(W.shape))
        args.append(W)
    return pl.pallas_call(
        body,
        grid=(_G,),
        in_specs=in_specs,
        out_specs=[_row(CO)] * nsplit,
        out_shape=[jax.ShapeDtypeStruct((_NPAD, CO), jnp.float32)] * nsplit,
    )(*args)


def _fin_call(pre, sums, g, bt, Wl, bl):
    F = pre.shape[1]
    FO = Wl.shape[1]

    def body(pre_ref, sums_ref, g_ref, bt_ref, W_ref, b_ref, out_ref):
        h = _bn_leaky(pre_ref[...], sums_ref, g_ref, bt_ref)
        out_ref[...] = jnp.dot(
            h, W_ref[...], preferred_element_type=jnp.float32) + b_ref[...]

    return pl.pallas_call(
        body,
        grid=(_G,),
        in_specs=[_row(F), _full((8, F)), _full((1, F)), _full((1, F)),
                  _full((F, FO)), _full((1, FO))],
        out_specs=_row(FO),
        out_shape=jax.ShapeDtypeStruct((_NPAD, FO), jnp.float32),
    )(pre, sums, g, bt, Wl, bl)


def kernel(x, edge_index, W0, b0, g0, bt0, W1, b1, g1, bt1, W2, b2, g2, bt2,
           W3, b3, g3, bt3, W4, b4, g4, bt4, Wlin, blin):
    f32 = jnp.float32
    src = edge_index[0]
    dst = edge_index[1]
    padlen = _EPAD - _E
    pad = jnp.full((padlen,), _N, jnp.int32)
    src2 = jnp.concatenate([src, pad]).reshape(_EPAD // 128, 128)
    dst2 = jnp.concatenate([dst, pad]).reshape(_EPAD // 128, 128)

    x_p = jnp.zeros((_NPAD, 8), f32).at[:_N, :6].set(x)
    node_mask = (jnp.arange(_NPAD) < _N).astype(f32)[:, None]
    zeros = {C: jnp.zeros((_NPAD, C), f32) for C in (1, 8, 16, 32)}

    W0p = jnp.zeros((8, 32), f32).at[:6].set(W0)
    Wlp = jnp.zeros((16, 8), f32).at[:, :3].set(Wlin)
    blp = jnp.zeros((1, 8), f32).at[0, :3].set(blin)
    r2 = lambda v: v.reshape(1, -1)

    def agg(u_chunks, C):
        k = _make_agg(C)
        return [k(u, src2, dst2, zeros[C]) for u in u_chunks]

    cnt, = agg([node_mask], 1)
    dis, u0 = _disk_call(cnt, x_p)

    pre, sums = _q_call(agg([u0], 8), [u0], dis, r2(b0), W0p)
    u1 = _p_call(pre, sums, r2(g0), r2(bt0), dis, 32)

    pre, sums = _q_call(agg(u1, 32), u1, dis, r2(b1), W1)
    u2 = _p_call(pre, sums, r2(g1), r2(bt1), dis, 32)

    pre, sums = _q_call(agg(u2, 32), u2, dis, r2(b2), W2)

    u3 = _p_call(pre, sums, r2(g2), r2(bt2), dis, 32, W3)
    pre, sums = _q_call(agg(u3, 32), u3, dis, r2(b3))

    u4 = _p_call(pre, sums, r2(g3), r2(bt3), dis, 16, W4)
    pre, sums = _q_call(agg(u4, 16), u4, dis, r2(b4))

    out = _fin_call(pre, sums, r2(g4), r2(bt4), Wlp, blp)
    return out[:_N, :3]
